# Initial kernel scaffold; baseline (speedup 1.0000x reference)
import jax
import jax.numpy as jnp
from jax.experimental import pallas as pl


def kernel(x, y, mean, sigma):
    raise NotImplementedError("write your pallas kernel here")


# speedup vs baseline: 1.0063x; 1.0063x over previous
import jax
import jax.numpy as jnp
import numpy as np
from jax.experimental import pallas as pl
from jax.experimental.pallas import tpu as pltpu

_B = 16384
_F = 512
_BLK_ROWS = 256

_KS0 = np.uint32(0)
_KS1 = np.uint32(1)
_KS2 = np.uint32(0x1BD11BDA) ^ _KS0 ^ _KS1
_KS = (_KS0, _KS1, _KS2)
_ROTATIONS = ((13, 15, 26, 6), (17, 29, 16, 24))

_LO = np.float32(np.nextafter(np.float32(-1.0), np.float32(0.0)))
_RANGE = np.float32(np.float32(1.0) - _LO)
_SQRT2 = np.float32(np.sqrt(2.0))


def _rotl(v, r):
    return (v << np.uint32(r)) | (v >> np.uint32(32 - r))


def _threefry2x32_xored(c1):
    x0 = jnp.zeros_like(c1) + _KS0
    x1 = c1 + _KS1
    for i in range(5):
        for r in _ROTATIONS[i % 2]:
            x0 = x0 + x1
            x1 = _rotl(x1, r)
            x1 = x0 ^ x1
        x0 = x0 + _KS[(i + 1) % 3]
        x1 = x1 + _KS[(i + 2) % 3] + np.uint32(i + 1)
    return x0 ^ x1


def _bits_to_normal(bits):
    fb = pltpu.bitcast((bits >> np.uint32(9)) | np.uint32(0x3F800000),
                       jnp.float32)
    u = fb - np.float32(1.0)
    x = jnp.maximum(_LO, u * _RANGE + _LO)
    w = -jnp.log1p(-(x * x))
    w_small = w - np.float32(2.5)
    p_s = np.float32(2.81022636e-08)
    for c in (3.43273939e-07, -3.5233877e-06, -4.39150654e-06, 0.00021858087,
              -0.00125372503, -0.00417768164, 0.246640727, 1.50140941):
        p_s = np.float32(c) + p_s * w_small
    w_big = jnp.sqrt(w) - np.float32(3.0)
    p_b = np.float32(-0.000200214257)
    for c in (0.000100950558, 0.00134934322, -0.00367342844, 0.00573950773,
              -0.0076224613, 0.00943887047, 1.00167406, 2.83297682):
        p_b = np.float32(c) + p_b * w_big
    p = jnp.where(w < np.float32(5.0), p_s, p_b)
    return _SQRT2 * (p * x)


def _noise_kernel(y_ref, mu_ref, sg_ref, o_ref):
    j = pl.program_id(0)
    base = (j * np.uint32(_BLK_ROWS * _F)).astype(jnp.uint32)
    row = jax.lax.broadcasted_iota(jnp.uint32, (_BLK_ROWS, _F), 0)
    col = jax.lax.broadcasted_iota(jnp.uint32, (_BLK_ROWS, _F), 1)
    idx = base + row * np.uint32(_F) + col
    z = _bits_to_normal(_threefry2x32_xored(idx))
    scale = jnp.exp(sg_ref[...])
    o_ref[...] = y_ref[...] + (z * scale + mu_ref[...])


def kernel(x, y, mean, sigma):
    del x
    mu = mean.reshape(1, _F)
    sg = sigma.reshape(1, _F)
    return pl.pallas_call(
        _noise_kernel,
        grid=(_B // _BLK_ROWS,),
        in_specs=[
            pl.BlockSpec((_BLK_ROWS, _F), lambda j: (j, 0)),
            pl.BlockSpec((1, _F), lambda j: (0, 0)),
            pl.BlockSpec((1, _F), lambda j: (0, 0)),
        ],
        out_specs=pl.BlockSpec((_BLK_ROWS, _F), lambda j: (j, 0)),
        out_shape=jax.ShapeDtypeStruct((_B, _F), jnp.float32),
        compiler_params=pltpu.CompilerParams(
            dimension_semantics=("arbitrary",),
        ),
    )(y, mu, sg)


# speedup vs baseline: 1.1597x; 1.1524x over previous
import jax
import jax.numpy as jnp
import numpy as np
from jax.experimental import pallas as pl
from jax.experimental.pallas import tpu as pltpu

_B = 16384
_F = 512
_BLK_ROWS = 256

_KS0 = np.uint32(0)
_KS1 = np.uint32(1)
_KS2 = np.uint32(0x1BD11BDA) ^ _KS0 ^ _KS1
_KS = (_KS0, _KS1, _KS2)
_ROTATIONS = ((13, 15, 26, 6), (17, 29, 16, 24))

_LO = np.float32(np.nextafter(np.float32(-1.0), np.float32(0.0)))

_ERFINV_COEF = (
    np.float32(-0.0016894012), np.float32(0.021943273),
    np.float32(-0.10480872), np.float32(0.21541858),
    np.float32(-0.19750586), np.float32(0.4306608),
    np.float32(-0.025189253), np.float32(1.2554606),
)


def _rotl(v, r):
    return (v << np.uint32(r)) | (v >> np.uint32(32 - r))


def _threefry2x32_xored(c1):
    x1 = c1 + _KS1
    x0 = x1
    x1 = _rotl(x1, 13) ^ x0
    for r in (15, 26, 6):
        x0 = x0 + x1
        x1 = _rotl(x1, r)
        x1 = x0 ^ x1
    x0 = x0 + _KS[1]
    x1 = x1 + _KS[2] + np.uint32(1)
    for i in range(1, 5):
        for r in _ROTATIONS[i % 2]:
            x0 = x0 + x1
            x1 = _rotl(x1, r)
            x1 = x0 ^ x1
        x0 = x0 + _KS[(i + 1) % 3]
        x1 = x1 + _KS[(i + 2) % 3] + np.uint32(i + 1)
    return x0 ^ x1


def _bits_to_normal(bits):
    fb = pltpu.bitcast((bits >> np.uint32(9)) | np.uint32(0x40000000),
                       jnp.float32)
    x = jnp.maximum(_LO, fb - np.float32(3.0))
    t = jnp.sqrt(-jnp.log1p(-(x * x)))
    p = _ERFINV_COEF[0]
    for c in _ERFINV_COEF[1:]:
        p = c + p * t
    return p * x


def _noise_kernel(y_ref, mu_ref, sg_ref, o_ref):
    j = pl.program_id(0)
    base = (j * np.uint32(_BLK_ROWS * _F)).astype(jnp.uint32)
    row = jax.lax.broadcasted_iota(jnp.uint32, (_BLK_ROWS, _F), 0)
    col = jax.lax.broadcasted_iota(jnp.uint32, (_BLK_ROWS, _F), 1)
    idx = base + row * np.uint32(_F) + col
    z = _bits_to_normal(_threefry2x32_xored(idx))
    scale = jnp.exp(sg_ref[...])
    o_ref[...] = y_ref[...] + (z * scale + mu_ref[...])


def kernel(x, y, mean, sigma):
    del x
    mu = mean.reshape(1, _F)
    sg = sigma.reshape(1, _F)
    return pl.pallas_call(
        _noise_kernel,
        grid=(_B // _BLK_ROWS,),
        in_specs=[
            pl.BlockSpec((_BLK_ROWS, _F), lambda j: (j, 0)),
            pl.BlockSpec((1, _F), lambda j: (0, 0)),
            pl.BlockSpec((1, _F), lambda j: (0, 0)),
        ],
        out_specs=pl.BlockSpec((_BLK_ROWS, _F), lambda j: (j, 0)),
        out_shape=jax.ShapeDtypeStruct((_B, _F), jnp.float32),
        compiler_params=pltpu.CompilerParams(
            dimension_semantics=("parallel",),
        ),
    )(y, mu, sg)


# speedup vs baseline: 1.2947x; 1.1164x over previous
import jax
import jax.numpy as jnp
import numpy as np
from jax.experimental import pallas as pl
from jax.experimental.pallas import tpu as pltpu

_B = 16384
_F = 512
_BLK_ROWS = 512

_KS0 = np.uint32(0)
_KS1 = np.uint32(1)
_KS2 = np.uint32(0x1BD11BDA) ^ _KS0 ^ _KS1
_KS = (_KS0, _KS1, _KS2)
_ROTATIONS = ((13, 15, 26, 6), (17, 29, 16, 24))

_LO = np.float32(np.nextafter(np.float32(-1.0), np.float32(0.0)))

_ERFINV_COEF = (
    np.float32(0.0075027994), np.float32(-0.07508623),
    np.float32(0.20664303), np.float32(0.13447733),
    np.float32(0.07743702), np.float32(1.2431358),
)


def _rotl(v, r):
    return (v << np.uint32(r)) | (v >> np.uint32(32 - r))


def _threefry2x32_xored(c1):
    x1 = c1
    x0 = x1
    x1 = _rotl(x1, 13) ^ x0
    for r in (15, 26, 6):
        x0 = x0 + x1
        x1 = _rotl(x1, r)
        x1 = x0 ^ x1
    x0 = x0 + _KS[1]
    x1 = x1 + (_KS[2] + np.uint32(1))
    for i in range(1, 5):
        for r in _ROTATIONS[i % 2]:
            x0 = x0 + x1
            x1 = _rotl(x1, r)
            x1 = x0 ^ x1
        if (i + 1) % 3 != 0:
            x0 = x0 + _KS[(i + 1) % 3]
        x1 = x1 + (_KS[(i + 2) % 3] + np.uint32(i + 1))
    return x0 ^ x1


def _bits_to_normal(bits):
    fb = pltpu.bitcast((bits >> np.uint32(9)) | np.uint32(0x40000000),
                       jnp.float32)
    x = jnp.maximum(_LO, fb - np.float32(3.0))
    t = jnp.sqrt(jnp.log(np.float32(1.0) - x * x) * np.float32(-1.0))
    p = _ERFINV_COEF[0]
    for c in _ERFINV_COEF[1:]:
        p = c + p * t
    return p * x


def _noise_kernel(y_ref, mu_ref, sg_ref, o_ref):
    j = pl.program_id(0)
    base1 = (j * np.uint32(_BLK_ROWS * _F) + np.uint32(1)).astype(jnp.uint32)
    row = jax.lax.broadcasted_iota(jnp.uint32, (_BLK_ROWS, _F), 0)
    col = jax.lax.broadcasted_iota(jnp.uint32, (_BLK_ROWS, _F), 1)
    c1 = base1 + row * np.uint32(_F) + col
    z = _bits_to_normal(_threefry2x32_xored(c1))
    scale = jnp.exp(sg_ref[...])
    o_ref[...] = y_ref[...] + (z * scale + mu_ref[...])


def kernel(x, y, mean, sigma):
    del x
    mu = mean.reshape(1, _F)
    sg = sigma.reshape(1, _F)
    return pl.pallas_call(
        _noise_kernel,
        grid=(_B // _BLK_ROWS,),
        in_specs=[
            pl.BlockSpec((_BLK_ROWS, _F), lambda j: (j, 0)),
            pl.BlockSpec((1, _F), lambda j: (0, 0)),
            pl.BlockSpec((1, _F), lambda j: (0, 0)),
        ],
        out_specs=pl.BlockSpec((_BLK_ROWS, _F), lambda j: (j, 0)),
        out_shape=jax.ShapeDtypeStruct((_B, _F), jnp.float32),
        compiler_params=pltpu.CompilerParams(
            dimension_semantics=("parallel",),
        ),
    )(y, mu, sg)
